# Initial kernel scaffold; baseline (speedup 1.0000x reference)
import jax
import jax.numpy as jnp
from jax.experimental import pallas as pl


def kernel(x, edge_index, W1, b1, g1, be1, W2, b2, g2, be2, Wc, bc):
    raise NotImplementedError("write your pallas kernel here")


# speedup vs baseline: 9.2433x; 9.2433x over previous
import functools

import jax
import jax.numpy as jnp
from jax import lax
from jax.experimental import pallas as pl
from jax.experimental.pallas import tpu as pltpu
from jax.experimental.pallas import tpu_sc as plsc

NNODE = 10000
NEDGE = 160000
FDIM = 256
FHALF = 128

NCORE = 2
NSUB = 16
LANES = 16
CHUNK = 128
NCHUNK = 79
EPAD = NSUB * NCHUNK * CHUNK
ACC_ROWS = 10240
ROWS_PER_TILE = ACC_ROWS // NSUB
OUT_ROWS_PER_TILE = NNODE // NSUB

_SC_MESH = plsc.VectorSubcoreMesh(core_axis_name="c", subcore_axis_name="s")


_DEG_SPLIT = 40


@functools.partial(
    pl.kernel,
    out_type=(
        jax.ShapeDtypeStruct((ACC_ROWS, FHALF), jnp.float32),
        jax.ShapeDtypeStruct((ACC_ROWS, FHALF), jnp.float32),
    ),
    mesh=_SC_MESH,
    scratch_types=[
        pltpu.VMEM((NCHUNK, CHUNK), jnp.int32),
        pltpu.VMEM((CHUNK, FHALF), jnp.float32),
        pltpu.VMEM_SHARED((ACC_ROWS, FHALF), jnp.float32),
        pltpu.SemaphoreType.DMA,
    ],
)
def _sc_degree(dst_hbm, degA_hbm, degB_hbm, dstv, ones_v, acc, sem):
    cid = lax.axis_index("c")
    sid = lax.axis_index("s")
    pltpu.sync_copy(dst_hbm.at[sid], dstv)

    def _fill(val):
        def _f(r, _):
            def _st(c, _):
                ones_v[r, pl.ds(c * LANES, LANES)] = jnp.full((LANES,), val, jnp.float32)
                return 0
            lax.fori_loop(0, FHALF // LANES, _st, 0)
            return 0
        lax.fori_loop(0, CHUNK, _f, 0)

    _fill(0.0)
    base = sid * ROWS_PER_TILE

    def _zero(k, _):
        pltpu.sync_copy(ones_v, acc.at[pl.ds(base + k * CHUNK, CHUNK)])
        return 0

    lax.fori_loop(0, ROWS_PER_TILE // CHUNK, _zero, 0)
    _fill(1.0)
    plsc.subcore_barrier()

    def _body(j, _):
        pltpu.sync_copy(ones_v, acc.at[dstv.at[j]], add=True)
        return 0

    @pl.when(cid == 0)
    def _():
        lax.fori_loop(0, _DEG_SPLIT, _body, 0)

    @pl.when(cid == 1)
    def _():
        lax.fori_loop(_DEG_SPLIT, NCHUNK, _body, 0)

    plsc.subcore_barrier()

    obase = sid * ROWS_PER_TILE

    @pl.when(cid == 0)
    def _():
        pltpu.sync_copy(acc.at[pl.ds(obase, ROWS_PER_TILE)],
                        degA_hbm.at[pl.ds(obase, ROWS_PER_TILE)])

    @pl.when(cid == 1)
    def _():
        pltpu.sync_copy(acc.at[pl.ds(obase, ROWS_PER_TILE)],
                        degB_hbm.at[pl.ds(obase, ROWS_PER_TILE)])


@functools.partial(
    pl.kernel,
    out_type=(
        jax.ShapeDtypeStruct((ACC_ROWS, FHALF), jnp.float32),
        jax.ShapeDtypeStruct((ACC_ROWS, FHALF), jnp.float32),
    ),
    mesh=_SC_MESH,
    scratch_types=[
        pltpu.VMEM((NCHUNK, CHUNK), jnp.int32),
        pltpu.VMEM((NCHUNK, CHUNK), jnp.int32),
        pltpu.VMEM((CHUNK, FHALF), jnp.float32),
        pltpu.VMEM_SHARED((ACC_ROWS, FHALF), jnp.float32),
        pltpu.SemaphoreType.DMA,
    ],
)
def _sc_segsum(src_hbm, dst_hbm, tabA_hbm, tabB_hbm, outA_hbm, outB_hbm,
               srcv, dstv, rows_v, acc, sem):
    cid = lax.axis_index("c")
    sid = lax.axis_index("s")
    pltpu.sync_copy(src_hbm.at[sid], srcv)
    pltpu.sync_copy(dst_hbm.at[sid], dstv)

    def _fill0(r, _):
        def _st(c, _):
            rows_v[r, pl.ds(c * LANES, LANES)] = jnp.zeros((LANES,), jnp.float32)
            return 0
        lax.fori_loop(0, FHALF // LANES, _st, 0)
        return 0

    lax.fori_loop(0, CHUNK, _fill0, 0)
    base = sid * ROWS_PER_TILE

    def _zero(k, _):
        pltpu.sync_copy(rows_v, acc.at[pl.ds(base + k * CHUNK, CHUNK)])
        return 0

    lax.fori_loop(0, ROWS_PER_TILE // CHUNK, _zero, 0)
    plsc.subcore_barrier()

    def _run(tab_hbm):
        def _body(j, _):
            pltpu.async_copy(tab_hbm.at[srcv.at[j]], rows_v, sem).wait()
            pltpu.sync_copy(rows_v, acc.at[dstv.at[j]], add=True)
            return 0
        lax.fori_loop(0, NCHUNK, _body, 0)

    @pl.when(cid == 0)
    def _():
        _run(tabA_hbm)

    @pl.when(cid == 1)
    def _():
        _run(tabB_hbm)

    plsc.subcore_barrier()

    obase = sid * ROWS_PER_TILE

    @pl.when(cid == 0)
    def _():
        pltpu.sync_copy(acc.at[pl.ds(obase, ROWS_PER_TILE)],
                        outA_hbm.at[pl.ds(obase, ROWS_PER_TILE)])

    @pl.when(cid == 1)
    def _():
        pltpu.sync_copy(acc.at[pl.ds(obase, ROWS_PER_TILE)],
                        outB_hbm.at[pl.ds(obase, ROWS_PER_TILE)])


RBLK = 1000
GRID = NNODE // RBLK


def _dis_of(degA, degB):
    deg = degA[:, 0:1] + degB[:, 0:1] + 1.0
    return lax.rsqrt(jnp.maximum(deg, 1.0))


def _tc1_body(x_ref, w1_ref, degA_ref, degB_ref, hA_ref, hB_ref):
    dis = _dis_of(degA_ref[...], degB_ref[...])
    h = jnp.dot(x_ref[...], w1_ref[...], preferred_element_type=jnp.float32)
    hp = h * dis
    hA_ref[...] = hp[:, :FHALF]
    hB_ref[...] = hp[:, FHALF:]


def _tc2_body(hA_ref, hB_ref, aA_ref, aB_ref, degA_ref, degB_ref,
              w2_ref, b1_ref, g1_ref, be1_ref, oA_ref, oB_ref):
    dis = _dis_of(degA_ref[...], degB_ref[...])
    hp = jnp.concatenate([hA_ref[...], hB_ref[...]], axis=1)
    acc = jnp.concatenate([aA_ref[...], aB_ref[...]], axis=1)
    conv = (hp + acc) * dis + b1_ref[...]
    act = jnp.maximum(conv * g1_ref[...] + be1_ref[...], 0.0)
    h2 = jnp.dot(act, w2_ref[...], preferred_element_type=jnp.float32) * dis
    oA_ref[...] = h2[:, :FHALF]
    oB_ref[...] = h2[:, FHALF:]


def _tc3_body(hA_ref, hB_ref, aA_ref, aB_ref, degA_ref, degB_ref,
              wc_ref, b2_ref, g2_ref, be2_ref, bc_ref, out_ref):
    dis = _dis_of(degA_ref[...], degB_ref[...])
    hp = jnp.concatenate([hA_ref[...], hB_ref[...]], axis=1)
    acc = jnp.concatenate([aA_ref[...], aB_ref[...]], axis=1)
    conv = (hp + acc) * dis + b2_ref[...]
    z = conv * g2_ref[...] + be2_ref[...]
    out_ref[...] = jnp.dot(z, wc_ref[...], preferred_element_type=jnp.float32) + bc_ref[...]


def _row_spec(cols):
    return pl.BlockSpec((RBLK, cols), lambda i: (i, 0))


def _full_spec(shape):
    return pl.BlockSpec(shape, lambda i: (0, 0))


def kernel(x, edge_index, W1, b1, g1, be1, W2, b2, g2, be2, Wc, bc):
    src = edge_index[0]
    dst = edge_index[1]
    pad = EPAD - NEDGE
    src_p = jnp.concatenate([src, jnp.zeros((pad,), jnp.int32)])
    dst_p = jnp.concatenate([dst, jnp.full((pad,), NNODE, jnp.int32)])
    src_r = src_p.reshape(NSUB, NCHUNK, CHUNK)
    dst_r = dst_p.reshape(NSUB, NCHUNK, CHUNK)

    degA, degB = _sc_degree(dst_r)
    degA = degA[:NNODE]
    degB = degB[:NNODE]

    hA, hB = pl.pallas_call(
        _tc1_body,
        grid=(GRID,),
        in_specs=[_row_spec(FDIM), _full_spec((FDIM, FDIM)),
                  _row_spec(FHALF), _row_spec(FHALF)],
        out_specs=[_row_spec(FHALF), _row_spec(FHALF)],
        out_shape=[jax.ShapeDtypeStruct((NNODE, FHALF), jnp.float32)] * 2,
    )(x, W1, degA, degB)

    accA, accB = _sc_segsum(src_r, dst_r, hA, hB)
    accA = accA[:NNODE]
    accB = accB[:NNODE]

    b1r = b1.reshape(1, FDIM)
    g1r = g1.reshape(1, FDIM)
    be1r = be1.reshape(1, FDIM)
    h2A, h2B = pl.pallas_call(
        _tc2_body,
        grid=(GRID,),
        in_specs=[_row_spec(FHALF), _row_spec(FHALF),
                  _row_spec(FHALF), _row_spec(FHALF),
                  _row_spec(FHALF), _row_spec(FHALF),
                  _full_spec((FDIM, FDIM)),
                  _full_spec((1, FDIM)), _full_spec((1, FDIM)),
                  _full_spec((1, FDIM))],
        out_specs=[_row_spec(FHALF), _row_spec(FHALF)],
        out_shape=[jax.ShapeDtypeStruct((NNODE, FHALF), jnp.float32)] * 2,
    )(hA, hB, accA, accB, degA, degB, W2, b1r, g1r, be1r)

    acc2A, acc2B = _sc_segsum(src_r, dst_r, h2A, h2B)
    acc2A = acc2A[:NNODE]
    acc2B = acc2B[:NNODE]

    nout = Wc.shape[1]
    wc_p = jnp.zeros((FDIM, FHALF), jnp.float32).at[:, :nout].set(Wc)
    bc_p = jnp.zeros((1, FHALF), jnp.float32).at[0, :nout].set(bc)
    b2r = b2.reshape(1, FDIM)
    g2r = g2.reshape(1, FDIM)
    be2r = be2.reshape(1, FDIM)
    out = pl.pallas_call(
        _tc3_body,
        grid=(GRID,),
        in_specs=[_row_spec(FHALF), _row_spec(FHALF),
                  _row_spec(FHALF), _row_spec(FHALF),
                  _row_spec(FHALF), _row_spec(FHALF),
                  _full_spec((FDIM, FHALF)),
                  _full_spec((1, FDIM)), _full_spec((1, FDIM)),
                  _full_spec((1, FDIM)), _full_spec((1, FHALF))],
        out_specs=_row_spec(FHALF),
        out_shape=jax.ShapeDtypeStruct((NNODE, FHALF), jnp.float32),
    )(h2A, h2B, acc2A, acc2B, degA, degB, wc_p, b2r, g2r, be2r, bc_p)

    return out[:, :nout]
